# Initial kernel scaffold; baseline (speedup 1.0000x reference)
import jax
import jax.numpy as jnp
from jax.experimental import pallas as pl


def kernel(x, edge_index, w):
    raise NotImplementedError("write your pallas kernel here")


# speedup vs baseline: 2.9602x; 2.9602x over previous
import functools

import jax
import jax.numpy as jnp
from jax import lax
from jax.experimental import pallas as pl
from jax.experimental.pallas import tpu as pltpu
from jax.experimental.pallas import tpu_sc as plsc

NC = 2
NS = 16
L = 16
NW = NC * NS
ZB = 80
EK = 128

_GDN = lax.GatherDimensionNumbers(
    offset_dims=(), collapsed_slice_dims=(0,), start_index_map=(0,))


def _perm(v, idx):
  return lax.gather(v, idx[:, None], _GDN, (1,),
                    mode=lax.GatherScatterMode.PROMISE_IN_BOUNDS)


def _lane_sum(v):
  lanes = lax.iota(jnp.int32, L)
  for k in (1, 2, 4, 8):
    v = v + _perm(v, lanes ^ k)
  return v


def _bcast_lane(v, e):
  return _perm(v, jnp.full((L,), e, jnp.int32))


def _rsqrt_vec(v):
  i = lax.bitcast_convert_type(v, jnp.int32)
  i = jnp.int32(0x5F3759DF) - (i >> 1)
  y = lax.bitcast_convert_type(i, jnp.float32)
  for _ in range(3):
    y = y * (1.5 - 0.5 * v * y * y)
  return y


def _mesh():
  return plsc.VectorSubcoreMesh(core_axis_name="c", subcore_axis_name="s",
                                num_cores=NC, num_subcores=NS)


_CPARAMS = pltpu.CompilerParams(needs_layout_passes=False)


def _make_k1(n, e, d):
  iters = e // EK // NW
  zit = n // ZB // NS
  jv = d // L

  @functools.partial(
      pl.kernel,
      out_type=(
          jax.ShapeDtypeStruct((NC * n, d), jnp.float32),
          jax.ShapeDtypeStruct((NW * n,), jnp.float32),
      ),
      mesh=_mesh(),
      compiler_params=_CPARAMS,
      scratch_types=[
          pltpu.VMEM((EK,), jnp.int32),
          pltpu.VMEM((EK,), jnp.int32),
          pltpu.VMEM((EK,), jnp.float32),
          pltpu.VMEM((EK, d), jnp.float32),
          pltpu.VMEM((ZB,), jnp.int32),
          pltpu.VMEM((ZB, d), jnp.float32),
          pltpu.VMEM((n,), jnp.float32),
          pltpu.VMEM_SHARED((n, d), jnp.float32),
          pltpu.SemaphoreType.DMA,
      ],
  )
  def k1(x_hbm, src_hbm, dst_hbm, w_hbm, px_hbm, pw_hbm,
         srcv, dstv, wv, rows, zidx, dbuf, awacc, accs, sem):
    cid = lax.axis_index("c")
    sid = lax.axis_index("s")
    wid = sid * NC + cid
    lanes = lax.iota(jnp.int32, L)

    def zdbuf(r, _):
      for j in range(jv):
        dbuf[r, pl.ds(j * L, L)] = jnp.zeros((L,), jnp.float32)
      return 0
    lax.fori_loop(0, ZB, zdbuf, 0)

    def zaw(r, _):
      awacc[pl.ds(r * L, L)] = jnp.zeros((L,), jnp.float32)
      return 0
    lax.fori_loop(0, n // L, zaw, 0)

    def set_zidx(base):
      for g in range(ZB // L):
        zidx[pl.ds(g * L, L)] = base + g * L + lanes

    def zchunk(z, _):
      base = (z * NS + sid) * ZB
      set_zidx(base)
      pltpu.sync_copy(dbuf, accs.at[zidx])
      return 0
    lax.fori_loop(0, zit, zchunk, 0)
    plsc.subcore_barrier()

    def chunk_body(i, _):
      off = (i * NW + wid) * EK
      pltpu.sync_copy(src_hbm.at[pl.ds(off, EK)], srcv)
      pltpu.sync_copy(dst_hbm.at[pl.ds(off, EK)], dstv)
      pltpu.sync_copy(w_hbm.at[pl.ds(off, EK)], wv)
      pltpu.async_copy(x_hbm.at[srcv], rows, sem).wait()

      for g in range(EK // L):
        awv = 1.0 - wv[pl.ds(g * L, L)]
        dst16 = dstv[pl.ds(g * L, L)]

        def edge_body(e16, _):
          erow = g * L + e16
          awb = _bcast_lane(awv, e16)
          for j in range(jv):
            sl = pl.ds(j * L, L)
            rows[erow, sl] = rows[erow, sl] * awb
          plsc.addupdate_scatter(awacc, [dst16], awv, mask=lanes == e16)
          return 0
        lax.fori_loop(0, L, edge_body, 0)

      pltpu.sync_copy(rows, accs.at[dstv], add=True)
      return 0
    lax.fori_loop(0, iters, chunk_body, 0)

    plsc.subcore_barrier()

    def dchunk(z, _):
      base = (z * NS + sid) * ZB
      set_zidx(base)
      pltpu.async_copy(accs.at[zidx], dbuf, sem).wait()
      pltpu.sync_copy(dbuf, px_hbm.at[pl.ds(cid * n + base, ZB)])
      return 0
    lax.fori_loop(0, zit, dchunk, 0)

    pltpu.sync_copy(awacc, pw_hbm.at[pl.ds(wid * n, n)])

  return k1


def _make_k2(n, d):
  iters = n // ZB // NW
  jv = d // L

  @functools.partial(
      pl.kernel,
      out_type=(
          jax.ShapeDtypeStruct((n, d), jnp.float32),
          jax.ShapeDtypeStruct((n,), jnp.float32),
      ),
      mesh=_mesh(),
      compiler_params=_CPARAMS,
      scratch_types=[
          pltpu.VMEM((ZB, d), jnp.float32),
          pltpu.VMEM((ZB, d), jnp.float32),
          pltpu.VMEM((ZB, d), jnp.float32),
          pltpu.VMEM((NW * ZB,), jnp.float32),
          pltpu.VMEM((ZB, d), jnp.float32),
          pltpu.VMEM((ZB,), jnp.float32),
      ],
  )
  def k2(x_hbm, px_hbm, pw_hbm, nx_hbm, rinv_hbm,
         xv, p0v, p1v, pwv, outv, rv):
    cid = lax.axis_index("c")
    sid = lax.axis_index("s")
    wid = sid * NC + cid
    lanes = lax.iota(jnp.int32, L)

    def chunk_body(i, _):
      ro = (i * NW + wid) * ZB
      sl_rows = pl.ds(ro, ZB)
      pltpu.sync_copy(x_hbm.at[sl_rows], xv)
      pltpu.sync_copy(px_hbm.at[pl.ds(ro, ZB)], p0v)
      pltpu.sync_copy(px_hbm.at[pl.ds(n + ro, ZB)], p1v)
      for t in range(NW):
        pltpu.sync_copy(pw_hbm.at[pl.ds(t * n + ro, ZB)],
                        pwv.at[pl.ds(t * ZB, ZB)])

      for g in range(ZB // L):
        saw = jnp.zeros((L,), jnp.float32)
        for t in range(NW):
          saw = saw + pwv[pl.ds(t * ZB + g * L, L)]

        def node_body(e16, rpack):
          r = g * L + e16
          den = 1.0 + _bcast_lane(saw, e16)
          ss = jnp.zeros((L,), jnp.float32)
          for j in range(jv):
            sl = pl.ds(j * L, L)
            num = (xv[r, sl] + p0v[r, sl] + p1v[r, sl]) / den
            outv[r, sl] = num
            ss = ss + num * num
          rr = jnp.minimum(_rsqrt_vec(_lane_sum(ss)), 1e8)
          return jnp.where(lanes == e16, rr, rpack)
        rpack = lax.fori_loop(0, L, node_body, jnp.zeros((L,), jnp.float32))
        rv[pl.ds(g * L, L)] = rpack

      pltpu.sync_copy(outv, nx_hbm.at[sl_rows])
      pltpu.sync_copy(rv, rinv_hbm.at[sl_rows])
      return 0
    lax.fori_loop(0, iters, chunk_body, 0)

  return k2


def _make_k3(n, e, d):
  iters = e // EK // NW
  jv = d // L

  @functools.partial(
      pl.kernel,
      out_type=jax.ShapeDtypeStruct((e,), jnp.float32),
      mesh=_mesh(),
      compiler_params=_CPARAMS,
      scratch_types=[
          pltpu.VMEM((EK,), jnp.int32),
          pltpu.VMEM((EK,), jnp.int32),
          pltpu.VMEM((EK,), jnp.float32),
          pltpu.VMEM((EK, d), jnp.float32),
          pltpu.VMEM((EK, d), jnp.float32),
          pltpu.VMEM((EK,), jnp.float32),
          pltpu.VMEM((n,), jnp.float32),
          pltpu.SemaphoreType.DMA,
          pltpu.SemaphoreType.DMA,
      ],
  )
  def k3(nx_hbm, rinv_hbm, src_hbm, dst_hbm, w_hbm, neww_hbm,
         srcv, dstv, wv, xs, xd, outv, rfull, sem1, sem2):
    cid = lax.axis_index("c")
    sid = lax.axis_index("s")
    wid = sid * NC + cid
    lanes = lax.iota(jnp.int32, L)

    pltpu.sync_copy(rinv_hbm, rfull)

    def chunk_body(i, _):
      off = (i * NW + wid) * EK
      pltpu.sync_copy(src_hbm.at[pl.ds(off, EK)], srcv)
      pltpu.sync_copy(dst_hbm.at[pl.ds(off, EK)], dstv)
      pltpu.sync_copy(w_hbm.at[pl.ds(off, EK)], wv)
      cp1 = pltpu.async_copy(nx_hbm.at[srcv], xs, sem1)
      cp2 = pltpu.async_copy(nx_hbm.at[dstv], xd, sem2)
      cp1.wait()
      cp2.wait()

      for g in range(EK // L):
        def edge_body(e16, dpack):
          erow = g * L + e16
          acc = jnp.zeros((L,), jnp.float32)
          for j in range(jv):
            sl = pl.ds(j * L, L)
            acc = acc + xs[erow, sl] * xd[erow, sl]
          dot = _lane_sum(acc)
          return jnp.where(lanes == e16, dot, dpack)
        dpack = lax.fori_loop(0, L, edge_body, jnp.zeros((L,), jnp.float32))

        gsl = pl.ds(g * L, L)
        rs = plsc.load_gather(rfull, [srcv[gsl]])
        rd = plsc.load_gather(rfull, [dstv[gsl]])
        cos = dpack * rs * rd
        cd = (1.0 - cos) * 0.5
        outv[gsl] = (wv[gsl] + cd) / (1.0 + cd)

      pltpu.sync_copy(outv, neww_hbm.at[pl.ds(off, EK)])
      return 0
    lax.fori_loop(0, iters, chunk_body, 0)

  return k3


def kernel(x, edge_index, w):
  n, d = x.shape
  e = w.shape[0]
  src = edge_index[0].astype(jnp.int32)
  dst = edge_index[1].astype(jnp.int32)

  nblk = ZB * NS
  eblk = EK * NW
  npad = -(-n // nblk) * nblk
  epad = -(-e // eblk) * eblk

  xp = jnp.pad(x, ((0, npad - n), (0, 0)))
  srcp = jnp.pad(src, (0, epad - e))
  dstp = jnp.pad(dst, (0, epad - e))
  wp = jnp.pad(w, (0, epad - e), constant_values=1.0)

  px, pw = _make_k1(npad, epad, d)(xp, srcp, dstp, wp)
  nx_p, rinv_p = _make_k2(npad, d)(xp, px, pw)
  neww_p = _make_k3(npad, epad, d)(nx_p, rinv_p, srcp, dstp, wp)
  return nx_p[:n], neww_p[:e]
